# baseline (speedup 1.0000x reference)
import functools

import jax
import jax.numpy as jnp
from jax import lax
from jax.experimental import pallas as pl
from jax.experimental.pallas import tpu as pltpu
from jax.experimental.pallas import tpu_sc as plsc

L = 16
NC = 2
NS = 16
NW = NC * NS
ROWS = 128
COLS = 32768
ROWS_PER_W = ROWS // NW
STEPS = COLS // L
UNROLL = 8
BIG = 1 << 30


def _fold(v, scratch, op):
    lane = lax.iota(jnp.int32, L)
    for k in (8, 4, 2, 1):
        scratch[...] = v
        g = plsc.load_gather(scratch, [lane ^ k])
        v = op(v, g)
    return v


def _find_top1(ref, scr_f, scr_i):
    def body(j, carry):
        m, s = carry
        for u in range(UNROLL):
            i = j * UNROLL + u
            v = ref[pl.ds(i * L, L)]
            c = v > m
            m = jnp.where(c, v, m)
            s = jnp.where(c, i, s)
        return m, s

    m0 = jnp.full((L,), -jnp.inf, dtype=jnp.float32)
    s0 = jnp.zeros((L,), dtype=jnp.int32)
    m, s = lax.fori_loop(0, STEPS // UNROLL, body, (m0, s0))
    lane = lax.iota(jnp.int32, L)
    col = s * L + lane
    mxv = _fold(m, scr_f, jnp.maximum)
    cand = jnp.where(m == mxv, col, BIG)
    return _fold(cand, scr_i, jnp.minimum)


def _mask_out(ref, iv):
    lane = lax.iota(jnp.int32, L)
    neg = jnp.full((L,), -jnp.inf, dtype=jnp.float32)
    plsc.store_scatter(ref, [iv], neg, mask=lane == 0)


def _body(x_hbm, out_hbm, buf0, buf1, outv, scr_f, scr_i, sem0, sem1):
    wid = lax.axis_index("s") * NC + lax.axis_index("c")
    base = wid * ROWS_PER_W
    bufs = (buf0, buf1)
    sems = (sem0, sem1)
    lane = lax.iota(jnp.int32, L)

    cps = [pltpu.async_copy(x_hbm.at[base], bufs[0], sems[0]), None]
    for rr in range(ROWS_PER_W):
        if rr + 1 < ROWS_PER_W:
            nb = (rr + 1) % 2
            cps[nb] = pltpu.async_copy(x_hbm.at[base + (rr + 1)], bufs[nb], sems[nb])
        cps[rr % 2].wait()
        ref = bufs[rr % 2]
        i1 = _find_top1(ref, scr_f, scr_i)
        _mask_out(ref, i1)
        i2 = _find_top1(ref, scr_f, scr_i)
        _mask_out(ref, i2)
        i3 = _find_top1(ref, scr_f, scr_i)
        res = jnp.where(lane == 0, i1, jnp.where(lane == 1, i2, jnp.where(lane == 2, i3, 0)))
        outv[rr] = res
    pltpu.sync_copy(outv, out_hbm.at[pl.ds(base, ROWS_PER_W)])


@jax.jit
def _topk3(x):
    mesh = plsc.VectorSubcoreMesh(core_axis_name="c", subcore_axis_name="s")
    run = pl.kernel(
        _body,
        out_type=jax.ShapeDtypeStruct((ROWS, L), jnp.int32),
        mesh=mesh,
        compiler_params=pltpu.CompilerParams(needs_layout_passes=False),
        scratch_types=[
            pltpu.VMEM((COLS,), jnp.float32),
            pltpu.VMEM((COLS,), jnp.float32),
            pltpu.VMEM((ROWS_PER_W, L), jnp.int32),
            pltpu.VMEM((L,), jnp.float32),
            pltpu.VMEM((L,), jnp.int32),
            pltpu.SemaphoreType.DMA,
            pltpu.SemaphoreType.DMA,
        ],
    )
    return run(x)[:, :3]


def kernel(x):
    return _topk3(x)


# speedup vs baseline: 1.2992x; 1.2992x over previous
import functools

import jax
import jax.numpy as jnp
from jax import lax
from jax.experimental import pallas as pl
from jax.experimental.pallas import tpu as pltpu
from jax.experimental.pallas import tpu_sc as plsc

L = 16
NC = 2
NS = 16
NW = NC * NS
ROWS = 128
COLS = 32768
ROWS_PER_W = ROWS // NW
STEPS = COLS // L
UNROLL = 8
BIG = 1 << 30


def _fold(v, scratch, op):
    lane = lax.iota(jnp.int32, L)
    for k in (8, 4, 2, 1):
        scratch[...] = v
        g = plsc.load_gather(scratch, [lane ^ k])
        v = op(v, g)
    return v


NACC = 4


def _find_top1(ref, scr_f, scr_i):
    def body(j, carry):
        ms = list(carry[:NACC])
        ss = list(carry[NACC:])
        for u in range(UNROLL):
            i = j * UNROLL + u
            a = u % NACC
            v = ref[pl.ds(i * L, L)]
            c = v > ms[a]
            ms[a] = jnp.maximum(ms[a], v)
            ss[a] = jnp.where(c, i, ss[a])
        return tuple(ms) + tuple(ss)

    m0 = jnp.full((L,), -jnp.inf, dtype=jnp.float32)
    s0 = jnp.zeros((L,), dtype=jnp.int32)
    carry = (m0,) * NACC + (s0,) * NACC
    carry = lax.fori_loop(0, STEPS // UNROLL, body, carry)
    ms = list(carry[:NACC])
    ss = list(carry[NACC:])

    lane = lax.iota(jnp.int32, L)

    def merge(ma, sa, mb, sb):
        ca = sa * L + lane
        cb = sb * L + lane
        c = (mb > ma) | ((mb == ma) & (cb < ca))
        return jnp.where(c, mb, ma), jnp.where(c, sb, sa)

    m01, s01 = merge(ms[0], ss[0], ms[1], ss[1])
    m23, s23 = merge(ms[2], ss[2], ms[3], ss[3])
    m, s = merge(m01, s01, m23, s23)

    col = s * L + lane
    mxv = _fold(m, scr_f, jnp.maximum)
    cand = jnp.where(m == mxv, col, BIG)
    return _fold(cand, scr_i, jnp.minimum)


def _mask_out(ref, iv):
    lane = lax.iota(jnp.int32, L)
    neg = jnp.full((L,), -jnp.inf, dtype=jnp.float32)
    plsc.store_scatter(ref, [iv], neg, mask=lane == 0)


def _body(x_hbm, out_hbm, buf0, buf1, outv, scr_f, scr_i, sem0, sem1):
    wid = lax.axis_index("s") * NC + lax.axis_index("c")
    base = wid * ROWS_PER_W
    bufs = (buf0, buf1)
    sems = (sem0, sem1)
    lane = lax.iota(jnp.int32, L)

    cps = [pltpu.async_copy(x_hbm.at[base], bufs[0], sems[0]), None]
    for rr in range(ROWS_PER_W):
        if rr + 1 < ROWS_PER_W:
            nb = (rr + 1) % 2
            cps[nb] = pltpu.async_copy(x_hbm.at[base + (rr + 1)], bufs[nb], sems[nb])
        cps[rr % 2].wait()
        ref = bufs[rr % 2]
        i1 = _find_top1(ref, scr_f, scr_i)
        _mask_out(ref, i1)
        i2 = _find_top1(ref, scr_f, scr_i)
        _mask_out(ref, i2)
        i3 = _find_top1(ref, scr_f, scr_i)
        res = jnp.where(lane == 0, i1, jnp.where(lane == 1, i2, jnp.where(lane == 2, i3, 0)))
        outv[rr] = res
    pltpu.sync_copy(outv, out_hbm.at[pl.ds(base, ROWS_PER_W)])


@jax.jit
def _topk3(x):
    mesh = plsc.VectorSubcoreMesh(core_axis_name="c", subcore_axis_name="s")
    run = pl.kernel(
        _body,
        out_type=jax.ShapeDtypeStruct((ROWS, L), jnp.int32),
        mesh=mesh,
        compiler_params=pltpu.CompilerParams(needs_layout_passes=False),
        scratch_types=[
            pltpu.VMEM((COLS,), jnp.float32),
            pltpu.VMEM((COLS,), jnp.float32),
            pltpu.VMEM((ROWS_PER_W, L), jnp.int32),
            pltpu.VMEM((L,), jnp.float32),
            pltpu.VMEM((L,), jnp.int32),
            pltpu.SemaphoreType.DMA,
            pltpu.SemaphoreType.DMA,
        ],
    )
    return run(x)[:, :3]


def kernel(x):
    return _topk3(x)


# speedup vs baseline: 1.6108x; 1.2398x over previous
import jax
import jax.numpy as jnp
from jax import lax
from jax.experimental import pallas as pl
from jax.experimental.pallas import tpu as pltpu
from jax.experimental.pallas import tpu_sc as plsc

L = 16
NC = 2
NS = 16
NW = NC * NS
ROWS = 128
COLS = 32768
ROWS_PER_W = ROWS // NW
STEPS = COLS // L
NBLK = 32
BSTEPS = STEPS // NBLK
UNROLL = 8
NACC = 4
BIG = 1 << 30


def _merge(ma, sa, mb, sb, lane):
    ca = sa * L + lane
    cb = sb * L + lane
    c = (mb > ma) | ((mb == ma) & (cb < ca))
    return jnp.where(c, mb, ma), jnp.where(c, sb, sa)


def _merge_accs(ms, ss, lane):
    m01, s01 = _merge(ms[0], ss[0], ms[1], ss[1], lane)
    m23, s23 = _merge(ms[2], ss[2], ms[3], ss[3], lane)
    return _merge(m01, s01, m23, s23, lane)


def _summarize_block(ref, b):
    def inner(j, carry):
        ms = list(carry[:NACC])
        ss = list(carry[NACC:])
        for u in range(UNROLL):
            g = b * BSTEPS + j * UNROLL + u
            v = ref[pl.ds(g * L, L)]
            a = u % NACC
            c = v > ms[a]
            ms[a] = jnp.maximum(ms[a], v)
            ss[a] = jnp.where(c, g, ss[a])
        return tuple(ms) + tuple(ss)

    m0 = jnp.full((L,), -jnp.inf, dtype=jnp.float32)
    s0 = jnp.zeros((L,), dtype=jnp.int32)
    carry = lax.fori_loop(0, BSTEPS // UNROLL, inner, (m0,) * NACC + (s0,) * NACC)
    lane = lax.iota(jnp.int32, L)
    return _merge_accs(list(carry[:NACC]), list(carry[NACC:]), lane)


def _scan_summaries(bmax_ref, bstep_ref):
    def inner(j, carry):
        ms = list(carry[:NACC])
        ss = list(carry[NACC:])
        for u in range(NACC):
            b = j * NACC + u
            mv = bmax_ref[pl.ds(b * L, L)]
            sv = bstep_ref[pl.ds(b * L, L)]
            c = mv > ms[u]
            ms[u] = jnp.maximum(ms[u], mv)
            ss[u] = jnp.where(c, sv, ss[u])
        return tuple(ms) + tuple(ss)

    m0 = jnp.full((L,), -jnp.inf, dtype=jnp.float32)
    s0 = jnp.zeros((L,), dtype=jnp.int32)
    carry = lax.fori_loop(0, NBLK // NACC, inner, (m0,) * NACC + (s0,) * NACC)
    lane = lax.iota(jnp.int32, L)
    return _merge_accs(list(carry[:NACC]), list(carry[NACC:]), lane)


def _fold(v, scratch, op):
    lane = lax.iota(jnp.int32, L)
    for k in (8, 4, 2, 1):
        scratch[...] = v
        g = plsc.load_gather(scratch, [lane ^ k])
        v = op(v, g)
    return v


def _body(x_hbm, out_hbm, buf0, buf1, bmax, bstep, outv, scr_f, scr_i, sem0, sem1):
    wid = lax.axis_index("s") * NC + lax.axis_index("c")
    base = wid * ROWS_PER_W
    bufs = (buf0, buf1)
    sems = (sem0, sem1)
    lane = lax.iota(jnp.int32, L)
    neg = jnp.full((L,), -jnp.inf, dtype=jnp.float32)

    cps = [pltpu.async_copy(x_hbm.at[base], bufs[0], sems[0]), None]
    for rr in range(ROWS_PER_W):
        if rr + 1 < ROWS_PER_W:
            nb = (rr + 1) % 2
            cps[nb] = pltpu.async_copy(x_hbm.at[base + (rr + 1)], bufs[nb], sems[nb])
        cps[rr % 2].wait()
        ref = bufs[rr % 2]

        def mainb(b, z, ref=ref):
            m, s = _summarize_block(ref, b)
            bmax[pl.ds(b * L, L)] = m
            bstep[pl.ds(b * L, L)] = s
            return z

        lax.fori_loop(0, NBLK, mainb, 0)

        def extract(p, res, ref=ref):
            m, s = _scan_summaries(bmax, bstep)
            col = s * L + lane
            mxv = _fold(m, scr_f, jnp.maximum)
            cand = jnp.where(m == mxv, col, BIG)
            iv = _fold(cand, scr_i, jnp.minimum)
            i1s = iv[0]
            plsc.store_scatter(ref, [iv], neg, mask=lane == 0)
            b1 = lax.shift_right_logical(i1s, 10)
            m2, s2 = _summarize_block(ref, b1)
            bmax[pl.ds(b1 * L, L)] = m2
            bstep[pl.ds(b1 * L, L)] = s2
            return jnp.where(lane == p, iv, res)

        res = lax.fori_loop(0, 3, extract, jnp.zeros((L,), dtype=jnp.int32))
        outv[rr] = res
    pltpu.sync_copy(outv, out_hbm.at[pl.ds(base, ROWS_PER_W)])


@jax.jit
def _topk3(x):
    mesh = plsc.VectorSubcoreMesh(core_axis_name="c", subcore_axis_name="s")
    run = pl.kernel(
        _body,
        out_type=jax.ShapeDtypeStruct((ROWS, L), jnp.int32),
        mesh=mesh,
        compiler_params=pltpu.CompilerParams(needs_layout_passes=False),
        scratch_types=[
            pltpu.VMEM((COLS,), jnp.float32),
            pltpu.VMEM((COLS,), jnp.float32),
            pltpu.VMEM((NBLK * L,), jnp.float32),
            pltpu.VMEM((NBLK * L,), jnp.int32),
            pltpu.VMEM((ROWS_PER_W, L), jnp.int32),
            pltpu.VMEM((L,), jnp.float32),
            pltpu.VMEM((L,), jnp.int32),
            pltpu.SemaphoreType.DMA,
            pltpu.SemaphoreType.DMA,
        ],
    )
    return run(x)[:, :3]


def kernel(x):
    return _topk3(x)
